# Initial kernel scaffold; baseline (speedup 1.0000x reference)
import jax
import jax.numpy as jnp
from jax.experimental import pallas as pl


def kernel(x, edge_index, W_gcn, b_gcn, W1, b1, W2, b2, W3, b3):
    raise NotImplementedError("write your pallas kernel here")


# speedup vs baseline: 11.2014x; 11.2014x over previous
import functools

import jax
import jax.numpy as jnp
from jax import lax
from jax.experimental import pallas as pl
from jax.experimental.pallas import tpu as pltpu
from jax.experimental.pallas import tpu_sc as plsc

N_NODES = 10000
D = 128
H1 = 256
H2 = 128
N_EDGES = 320000

NPAD = 10240
ZROW = 10000
TRASH = 10200
CHUNK = 128
NC, NS = 2, 16
NW = NC * NS
CPW = 80
EPW = CPW * CHUNK
EPAD = EPW * NW
RPS = NPAD // NS

def _mesh():
    return plsc.VectorSubcoreMesh(core_axis_name="c", subcore_axis_name="s")


def _sc_hist(dstm, ones128, zeros128):
    @functools.partial(
        pl.kernel,
        out_type=jax.ShapeDtypeStruct((2 * NPAD, D), jnp.float32),
        mesh=_mesh(),
        scratch_types=[
            pltpu.VMEM((CPW, CHUNK), jnp.int32),
            pltpu.VMEM((CHUNK, D), jnp.float32),
            pltpu.VMEM_SHARED((NPAD, D), jnp.float32),
        ],
    )
    def k(dst_hbm, ones_hbm, z_hbm, out_hbm, idx_v, ones_v, hist_sp):
        c = lax.axis_index("c")
        s = lax.axis_index("s")
        wid = s * NC + c
        pltpu.sync_copy(z_hbm, hist_sp.at[pl.ds(s * RPS, RPS)])
        pltpu.sync_copy(ones_hbm, ones_v)
        pltpu.sync_copy(dst_hbm.at[pl.ds(wid * CPW, CPW)], idx_v)
        plsc.subcore_barrier()

        @pl.loop(0, CPW)
        def _(i):
            pltpu.sync_copy(ones_v, hist_sp.at[idx_v.at[i]], add=True)

        plsc.subcore_barrier()
        pltpu.sync_copy(
            hist_sp.at[pl.ds(s * RPS, RPS)],
            out_hbm.at[pl.ds(c * NPAD + s * RPS, RPS)],
        )

    return k(dstm, ones128, zeros128)


def _sc_seg(y, srcm, dstm, zeros128):
    @functools.partial(
        pl.kernel,
        out_type=jax.ShapeDtypeStruct((2 * NPAD, D), jnp.float32),
        mesh=_mesh(),
        scratch_types=[
            pltpu.VMEM((CPW, CHUNK), jnp.int32),
            pltpu.VMEM((CPW, CHUNK), jnp.int32),
            pltpu.VMEM((CHUNK, D), jnp.float32),
            pltpu.VMEM_SHARED((NPAD, D), jnp.float32),
        ],
    )
    def k(y_hbm, src_hbm, dst_hbm, z_hbm, out_hbm, si_v, di_v, rows_v, acc_sp):
        c = lax.axis_index("c")
        s = lax.axis_index("s")
        wid = s * NC + c
        pltpu.sync_copy(z_hbm, acc_sp.at[pl.ds(s * RPS, RPS)])
        pltpu.sync_copy(src_hbm.at[pl.ds(wid * CPW, CPW)], si_v)
        pltpu.sync_copy(dst_hbm.at[pl.ds(wid * CPW, CPW)], di_v)
        plsc.subcore_barrier()

        @pl.loop(0, CPW)
        def _(i):
            pltpu.sync_copy(y_hbm.at[si_v.at[i]], rows_v)
            pltpu.sync_copy(rows_v, acc_sp.at[di_v.at[i]], add=True)

        plsc.subcore_barrier()
        pltpu.sync_copy(
            acc_sp.at[pl.ds(s * RPS, RPS)],
            out_hbm.at[pl.ds(c * NPAD + s * RPS, RPS)],
        )

    return k(y, srcm, dstm, zeros128)


def _tc_matmul(x_pad, W):
    R = 512

    def body(xr, wr, outr):
        outr[...] = jnp.dot(xr[...], wr[...], preferred_element_type=jnp.float32)

    return pl.pallas_call(
        body,
        grid=(NPAD // R,),
        in_specs=[
            pl.BlockSpec((R, D), lambda i: (i, 0)),
            pl.BlockSpec((D, D), lambda i: (0, 0)),
        ],
        out_specs=pl.BlockSpec((R, D), lambda i: (i, 0)),
        out_shape=jax.ShapeDtypeStruct((NPAD, D), jnp.float32),
    )(x_pad, W)


def _tc_scale(h0, h1, xw):
    R = 512

    def body(h0r, h1r, xwr, yr):
        deg = h0r[...][:, :1] + h1r[...][:, :1] + 1.0
        yr[...] = xwr[...] * lax.rsqrt(deg)

    return pl.pallas_call(
        body,
        grid=(NPAD // R,),
        in_specs=[
            pl.BlockSpec((R, D), lambda i: (i, 0)),
            pl.BlockSpec((R, D), lambda i: (i, 0)),
            pl.BlockSpec((R, D), lambda i: (i, 0)),
        ],
        out_specs=pl.BlockSpec((R, D), lambda i: (i, 0)),
        out_shape=jax.ShapeDtypeStruct((NPAD, D), jnp.float32),
    )(h0, h1, xw)


def _tc_head(p0, p1, h0, h1, xw, x_pad, bg, W1, bb1, W2, bb2, W3p, bb3):
    R = 1024

    def body(p0r, p1r, h0r, h1r, xwr, xr, bgr, w1r, b1r, w2r, b2r, w3r, b3r, outr):
        deg = h0r[...][:, :1] + h1r[...][:, :1] + 1.0
        dinv = lax.rsqrt(deg)
        gcn = (p0r[...] + p1r[...]) * dinv + xwr[...] / deg + bgr[...]
        h = jnp.maximum(gcn, 0.0) + xr[...]
        a1 = jnp.maximum(
            jnp.dot(h, w1r[...], preferred_element_type=jnp.float32) + b1r[...], 0.0
        )
        a2 = jnp.maximum(
            jnp.dot(a1, w2r[...], preferred_element_type=jnp.float32) + b2r[...], 0.0
        )
        outr[...] = jnp.dot(a2, w3r[...], preferred_element_type=jnp.float32) + b3r[...]

    row = lambda i: (i, 0)
    rep = lambda i: (0, 0)
    return pl.pallas_call(
        body,
        grid=(NPAD // R,),
        in_specs=[
            pl.BlockSpec((R, D), row),
            pl.BlockSpec((R, D), row),
            pl.BlockSpec((R, D), row),
            pl.BlockSpec((R, D), row),
            pl.BlockSpec((R, D), row),
            pl.BlockSpec((R, D), row),
            pl.BlockSpec((1, D), rep),
            pl.BlockSpec((D, H1), rep),
            pl.BlockSpec((1, H1), rep),
            pl.BlockSpec((H1, H2), rep),
            pl.BlockSpec((1, H2), rep),
            pl.BlockSpec((H2, D), rep),
            pl.BlockSpec((1, D), rep),
        ],
        out_specs=pl.BlockSpec((R, D), row),
        out_shape=jax.ShapeDtypeStruct((NPAD, D), jnp.float32),
    )(p0, p1, h0, h1, xw, x_pad, bg, W1, bb1, W2, bb2, W3p, bb3)


def kernel(x, edge_index, W_gcn, b_gcn, W1, b1, W2, b2, W3, b3):
    src = edge_index[0].astype(jnp.int32)
    dst = edge_index[1].astype(jnp.int32)
    npe = EPAD - N_EDGES
    srcm = jnp.concatenate(
        [src, jnp.full((npe,), ZROW, jnp.int32)]
    ).reshape(NW * CPW, CHUNK)
    dstm = jnp.concatenate(
        [dst, jnp.full((npe,), TRASH, jnp.int32)]
    ).reshape(NW * CPW, CHUNK)

    ones128 = jnp.ones((CHUNK, D), jnp.float32)
    zeros128 = jnp.zeros((RPS, D), jnp.float32)
    x_pad = jnp.pad(x, ((0, NPAD - N_NODES), (0, 0)))

    hist = _sc_hist(dstm, ones128, zeros128)
    h0, h1 = hist[:NPAD], hist[NPAD:]
    xw = _tc_matmul(x_pad, W_gcn)
    y = _tc_scale(h0, h1, xw)
    parts = _sc_seg(y, srcm, dstm, zeros128)

    W3p = jnp.pad(W3, ((0, 0), (0, D - 1)))
    bg = b_gcn.reshape(1, D)
    bb1 = b1.reshape(1, H1)
    bb2 = b2.reshape(1, H2)
    bb3 = jnp.pad(b3.reshape(1, 1), ((0, 0), (0, D - 1)))
    out_full = _tc_head(
        parts[:NPAD], parts[NPAD:], h0, h1, xw, x_pad, bg, W1, bb1, W2, bb2, W3p, bb3
    )
    return out_full[:N_NODES, :1]


# speedup vs baseline: 11.2102x; 1.0008x over previous
import functools

import jax
import jax.numpy as jnp
from jax import lax
from jax.experimental import pallas as pl
from jax.experimental.pallas import tpu as pltpu
from jax.experimental.pallas import tpu_sc as plsc

N_NODES = 10000
D = 128
H1 = 256
H2 = 128
N_EDGES = 320000

NPAD = 10240
ZROW = 10000
TRASH = 10200
CHUNK = 128
NC, NS = 2, 16
NW = NC * NS
CPW = 80
EPW = CPW * CHUNK
EPAD = EPW * NW
RPS = NPAD // NS

def _mesh():
    return plsc.VectorSubcoreMesh(core_axis_name="c", subcore_axis_name="s")


def _sc_hist(dstm, ones128, zeros128):
    @functools.partial(
        pl.kernel,
        out_type=jax.ShapeDtypeStruct((2 * NPAD, D), jnp.float32),
        mesh=_mesh(),
        scratch_types=[
            pltpu.VMEM((CPW, CHUNK), jnp.int32),
            pltpu.VMEM((CHUNK, D), jnp.float32),
            pltpu.VMEM_SHARED((NPAD, D), jnp.float32),
        ],
    )
    def k(dst_hbm, ones_hbm, z_hbm, out_hbm, idx_v, ones_v, hist_sp):
        c = lax.axis_index("c")
        s = lax.axis_index("s")
        wid = s * NC + c
        pltpu.sync_copy(z_hbm, hist_sp.at[pl.ds(s * RPS, RPS)])
        pltpu.sync_copy(ones_hbm, ones_v)
        pltpu.sync_copy(dst_hbm.at[pl.ds(wid * CPW, CPW)], idx_v)
        plsc.subcore_barrier()

        @pl.loop(0, CPW)
        def _(i):
            pltpu.sync_copy(ones_v, hist_sp.at[idx_v.at[i]], add=True)

        plsc.subcore_barrier()
        pltpu.sync_copy(
            hist_sp.at[pl.ds(s * RPS, RPS)],
            out_hbm.at[pl.ds(c * NPAD + s * RPS, RPS)],
        )

    return k(dstm, ones128, zeros128)


def _sc_seg(y, srcm, dstm, zeros128):
    @functools.partial(
        pl.kernel,
        out_type=jax.ShapeDtypeStruct((2 * NPAD, D), jnp.float32),
        mesh=_mesh(),
        scratch_types=[
            pltpu.VMEM((CPW, CHUNK), jnp.int32),
            pltpu.VMEM((CPW, CHUNK), jnp.int32),
            pltpu.VMEM((CHUNK, D), jnp.float32),
            pltpu.VMEM_SHARED((NPAD, D), jnp.float32),
        ],
    )
    def k(y_hbm, src_hbm, dst_hbm, z_hbm, out_hbm, si_v, di_v, rows_v, acc_sp):
        c = lax.axis_index("c")
        s = lax.axis_index("s")
        wid = s * NC + c
        pltpu.sync_copy(z_hbm, acc_sp.at[pl.ds(s * RPS, RPS)])
        pltpu.sync_copy(src_hbm.at[pl.ds(wid * CPW, CPW)], si_v)
        pltpu.sync_copy(dst_hbm.at[pl.ds(wid * CPW, CPW)], di_v)
        plsc.subcore_barrier()

        @pl.loop(0, CPW)
        def _(i):
            pltpu.sync_copy(y_hbm.at[si_v.at[i]], rows_v)
            pltpu.sync_copy(rows_v, acc_sp.at[di_v.at[i]], add=True)

        plsc.subcore_barrier()
        pltpu.sync_copy(
            acc_sp.at[pl.ds(s * RPS, RPS)],
            out_hbm.at[pl.ds(c * NPAD + s * RPS, RPS)],
        )

    return k(y, srcm, dstm, zeros128)


def _tc_matmul(x_pad, W):
    R = 512

    def body(xr, wr, outr):
        outr[...] = jnp.dot(xr[...], wr[...], preferred_element_type=jnp.float32)

    return pl.pallas_call(
        body,
        grid=(NPAD // R,),
        in_specs=[
            pl.BlockSpec((R, D), lambda i: (i, 0)),
            pl.BlockSpec((D, D), lambda i: (0, 0)),
        ],
        out_specs=pl.BlockSpec((R, D), lambda i: (i, 0)),
        out_shape=jax.ShapeDtypeStruct((NPAD, D), jnp.float32),
    )(x_pad, W)


def _tc_scale(h0, h1, xw):
    R = 512

    def body(h0r, h1r, xwr, yr):
        deg = h0r[...][:, :1] + h1r[...][:, :1] + 1.0
        yr[...] = xwr[...] * lax.rsqrt(deg)

    return pl.pallas_call(
        body,
        grid=(NPAD // R,),
        in_specs=[
            pl.BlockSpec((R, D), lambda i: (i, 0)),
            pl.BlockSpec((R, D), lambda i: (i, 0)),
            pl.BlockSpec((R, D), lambda i: (i, 0)),
        ],
        out_specs=pl.BlockSpec((R, D), lambda i: (i, 0)),
        out_shape=jax.ShapeDtypeStruct((NPAD, D), jnp.float32),
    )(h0, h1, xw)


def _tc_head(p0, p1, h0, h1, xw, x_pad, bg, W1, bb1, W2, bb2, W3p, bb3):
    R = 1024

    def body(p0r, p1r, h0r, h1r, xwr, xr, bgr, w1r, b1r, w2r, b2r, w3r, b3r, outr):
        deg = h0r[...][:, :1] + h1r[...][:, :1] + 1.0
        dinv = lax.rsqrt(deg)
        gcn = (p0r[...] + p1r[...]) * dinv + xwr[...] / deg + bgr[...]
        h = jnp.maximum(gcn, 0.0) + xr[...]
        a1 = jnp.maximum(
            jnp.dot(h, w1r[...], preferred_element_type=jnp.float32) + b1r[...], 0.0
        )
        a2 = jnp.maximum(
            jnp.dot(a1, w2r[...], preferred_element_type=jnp.float32) + b2r[...], 0.0
        )
        outr[...] = jnp.dot(a2, w3r[...], preferred_element_type=jnp.float32) + b3r[...]

    row = lambda i: (i, 0)
    rep = lambda i: (0, 0)
    return pl.pallas_call(
        body,
        grid=(NPAD // R,),
        in_specs=[
            pl.BlockSpec((R, D), row),
            pl.BlockSpec((R, D), row),
            pl.BlockSpec((R, D), row),
            pl.BlockSpec((R, D), row),
            pl.BlockSpec((R, D), row),
            pl.BlockSpec((R, D), row),
            pl.BlockSpec((1, D), rep),
            pl.BlockSpec((D, H1), rep),
            pl.BlockSpec((1, H1), rep),
            pl.BlockSpec((H1, H2), rep),
            pl.BlockSpec((1, H2), rep),
            pl.BlockSpec((H2, D), rep),
            pl.BlockSpec((1, D), rep),
        ],
        out_specs=pl.BlockSpec((R, D), row),
        out_shape=jax.ShapeDtypeStruct((NPAD, D), jnp.float32),
    )(p0, p1, h0, h1, xw, x_pad, bg, W1, bb1, W2, bb2, W3p, bb3)


def kernel(x, edge_index, W_gcn, b_gcn, W1, b1, W2, b2, W3, b3):
    src = edge_index[0].astype(jnp.int32)
    dst = edge_index[1].astype(jnp.int32)
    npe = EPAD - N_EDGES
    srcm = jnp.concatenate(
        [src, jnp.full((npe,), ZROW, jnp.int32)]
    ).reshape(NW * CPW, CHUNK)
    trash = N_NODES + (jnp.arange(npe, dtype=jnp.int32) % (NPAD - N_NODES))
    dstm = jnp.concatenate([dst, trash]).reshape(NW * CPW, CHUNK)

    ones128 = jnp.ones((CHUNK, D), jnp.float32)
    zeros128 = jnp.zeros((RPS, D), jnp.float32)
    x_pad = jnp.pad(x, ((0, NPAD - N_NODES), (0, 0)))

    hist = _sc_hist(dstm, ones128, zeros128)
    h0, h1 = hist[:NPAD], hist[NPAD:]
    xw = _tc_matmul(x_pad, W_gcn)
    y = _tc_scale(h0, h1, xw)
    parts = _sc_seg(y, srcm, dstm, zeros128)

    W3p = jnp.pad(W3, ((0, 0), (0, D - 1)))
    bg = b_gcn.reshape(1, D)
    bb1 = b1.reshape(1, H1)
    bb2 = b2.reshape(1, H2)
    bb3 = jnp.pad(b3.reshape(1, 1), ((0, 0), (0, D - 1)))
    out_full = _tc_head(
        parts[:NPAD], parts[NPAD:], h0, h1, xw, x_pad, bg, W1, bb1, W2, bb2, W3p, bb3
    )
    return out_full[:N_NODES, :1]
